# baseline (speedup 1.0000x reference)
import functools

import jax
import jax.numpy as jnp
from jax import lax
from jax.experimental import pallas as pl
from jax.experimental.pallas import tpu as pltpu
from jax.experimental.pallas import tpu_sc as plsc


def kernel(user, item, user_table, item_table, W, b):
    B = user.shape[0]
    D = user_table.shape[1]
    PACK = 128 // D

    info = plsc.get_sparse_core_info()
    NC, NS, L = info.num_cores, info.num_subcores, info.num_lanes
    NW = NC * NS
    b_per_w = B // NW
    CH = 128
    n_chunks = b_per_w // CH
    n_groups = CH // L

    user_r = user.reshape(NW, b_per_w)
    item_r = item.reshape(NW, b_per_w)
    ut128 = user_table.reshape(-1, 128)
    it128 = item_table.reshape(-1, 128)
    wb = jnp.concatenate(
        [jnp.repeat(W.reshape(-1), L), jnp.broadcast_to(b, (L,))])

    mesh = plsc.VectorSubcoreMesh(core_axis_name="c", subcore_axis_name="s")

    @functools.partial(
        pl.kernel,
        mesh=mesh,
        compiler_params=pltpu.CompilerParams(needs_layout_passes=False),
        out_type=jax.ShapeDtypeStruct((B,), jnp.float32),
        scratch_types=[
            pltpu.VMEM((b_per_w,), jnp.int32),
            pltpu.VMEM((b_per_w,), jnp.int32),
            pltpu.VMEM((b_per_w,), jnp.int32),
            pltpu.VMEM((b_per_w,), jnp.int32),
            pltpu.VMEM((b_per_w,), jnp.int32),
            pltpu.VMEM((b_per_w,), jnp.int32),
            pltpu.VMEM((CH, 128), jnp.float32),
            pltpu.VMEM((CH, 128), jnp.float32),
            pltpu.VMEM((b_per_w,), jnp.float32),
            pltpu.VMEM((D * 16 + 16,), jnp.float32),
            pltpu.SemaphoreType.DMA,
        ],
    )
    def gmf_sc(user_hbm, item_hbm, utab_hbm, itab_hbm, wb_hbm, out_hbm,
               idx_u, idx_i, ln_u, ln_i, of_u, of_i, rows_u, rows_i,
               acc, wb_v, sem):
        wid = lax.axis_index("s") * NC + lax.axis_index("c")
        base = wid * b_per_w

        pltpu.sync_copy(wb_hbm, wb_v)
        pltpu.sync_copy(user_hbm.at[wid], idx_u)
        pltpu.sync_copy(item_hbm.at[wid], idx_i)

        def prep(q, carry):
            s = pl.ds(q * L, L)
            uv = idx_u[s]
            iv = idx_i[s]
            ln_u[s] = uv >> 2
            ln_i[s] = iv >> 2
            of_u[s] = (uv & (PACK - 1)) * D
            of_i[s] = (iv & (PACK - 1)) * D
            return carry

        lax.fori_loop(0, b_per_w // L, prep, 0)

        bias = wb_v[pl.ds(D * L, L)]
        lanes = lax.iota(jnp.int32, L)

        for c in range(n_chunks):
            cu = pltpu.async_copy(
                utab_hbm.at[ln_u.at[pl.ds(c * CH, CH)]], rows_u, sem)
            ci = pltpu.async_copy(
                itab_hbm.at[ln_i.at[pl.ds(c * CH, CH)]], rows_i, sem)
            cu.wait()
            ci.wait()

            def group(g, carry, c=c):
                rvec = lanes + g * L
                ou = of_u[pl.ds(c * CH + g * L, L)]
                oi = of_i[pl.ds(c * CH + g * L, L)]
                s = bias
                for d in range(D):
                    gu = plsc.load_gather(rows_u, [rvec, ou + d])
                    gv = plsc.load_gather(rows_i, [rvec, oi + d])
                    s = s + gu * gv * wb_v[pl.ds(d * L, L)]
                acc[pl.ds(c * CH + g * L, L)] = s
                return carry

            lax.fori_loop(0, n_groups, group, 0)

        pltpu.sync_copy(acc, out_hbm.at[pl.ds(base, b_per_w)])

    return gmf_sc(user_r, item_r, ut128, it128, wb)


# speedup vs baseline: 3.1684x; 3.1684x over previous
import functools

import jax
import jax.numpy as jnp
from jax import lax
from jax.experimental import pallas as pl
from jax.experimental.pallas import tpu as pltpu
from jax.experimental.pallas import tpu_sc as plsc


def kernel(user, item, user_table, item_table, W, b):
    B = user.shape[0]
    D = user_table.shape[1]
    NA = 4
    L = 16

    info = plsc.get_sparse_core_info()
    NC, NS = info.num_cores, info.num_subcores
    NW = NC * NS
    b_per_w = B // NW
    NG = b_per_w // L
    QR = 4

    user_r = user.reshape(NW, b_per_w)
    item_r = item.reshape(NW, b_per_w)
    tut3 = user_table.T.reshape(NA, 8, user_table.shape[0])
    tit3 = item_table.T.reshape(NA, 8, item_table.shape[0])
    wb = jnp.concatenate([W.reshape(-1), jnp.broadcast_to(b, (L,))])

    mesh = plsc.VectorSubcoreMesh(core_axis_name="c", subcore_axis_name="s")

    @functools.partial(
        pl.kernel,
        mesh=mesh,
        compiler_params=pltpu.CompilerParams(needs_layout_passes=False),
        out_type=jax.ShapeDtypeStruct((B,), jnp.float32),
        scratch_types=[
            pltpu.VMEM((b_per_w,), jnp.int32),
            pltpu.VMEM((b_per_w,), jnp.int32),
            pltpu.VMEM((2, QR, NA, 8, 128), jnp.float32),
            pltpu.VMEM((2, QR, NA, 8, 128), jnp.float32),
            pltpu.VMEM((L * L,), jnp.float32),
            pltpu.VMEM((b_per_w,), jnp.float32),
            pltpu.VMEM((48,), jnp.float32),
            pltpu.SemaphoreType.DMA,
            pltpu.SemaphoreType.DMA,
        ],
    )
    def gmf_sc(user_hbm, item_hbm, utab_hbm, itab_hbm, wb_hbm, out_hbm,
               idx_u, idx_i, ublk, vblk, tbuf, acc, wb_v, sem0, sem1):
        sems = (sem0, sem1)
        wid = lax.axis_index("s") * NC + lax.axis_index("c")
        base = wid * b_per_w

        pltpu.sync_copy(wb_hbm, wb_v)
        pltpu.sync_copy(user_hbm.at[wid], idx_u)
        pltpu.sync_copy(item_hbm.at[wid], idx_i)

        w0 = wb_v[pl.ds(0, L)]
        w1 = wb_v[pl.ds(L, L)]
        bias = wb_v[pl.ds(2 * L, L)]
        lanes = lax.iota(jnp.int32, L)
        a_lo = lanes >> 3
        a_hi = a_lo + 2
        i_id = lanes & 7

        def issue(slot, k, u, v):
            bu = (u >> 7) * 128
            bv = (v >> 7) * 128
            pltpu.async_copy(
                utab_hbm.at[:, :, pl.ds(bu, 128)], ublk.at[slot, k],
                sems[slot])
            pltpu.async_copy(
                itab_hbm.at[:, :, pl.ds(bv, 128)], vblk.at[slot, k],
                sems[slot])

        def drain(slot, k):
            pltpu.make_async_copy(
                utab_hbm.at[:, :, pl.ds(0, 128)], ublk.at[slot, k],
                sems[slot]).wait()
            pltpu.make_async_copy(
                itab_hbm.at[:, :, pl.ds(0, 128)], vblk.at[slot, k],
                sems[slot]).wait()

        vu0 = idx_u[pl.ds(0, L)]
        vi0 = idx_i[pl.ds(0, L)]
        for k in range(QR):
            issue(0, k, vu0[k], vi0[k])

        def group(g, carry):
            vu = idx_u[pl.ds(g * L, L)]
            vi = idx_i[pl.ds(g * L, L)]
            vu_n = idx_u[pl.ds((g + 1) * L - b_per_w * (g // (NG - 1)), L)]
            vi_n = idx_i[pl.ds((g + 1) * L - b_per_w * (g // (NG - 1)), L)]
            for p in range(L // QR):
                slot = p & 1
                for k in range(QR):
                    drain(slot, k)
                for k in range(QR):
                    kk = (p + 1) * QR + k
                    if kk < L:
                        issue(slot ^ 1, k, vu[kk], vi[kk])
                    else:
                        issue(slot ^ 1, k, vu_n[kk - L], vi_n[kk - L])
                for k in range(QR):
                    u = vu[p * QR + k]
                    v = vi[p * QR + k]
                    ju = jnp.full((L,), u & 127, jnp.int32)
                    jv = jnp.full((L,), v & 127, jnp.int32)
                    sk = jnp.full((L,), slot, jnp.int32)
                    kk16 = jnp.full((L,), k, jnp.int32)
                    u0 = plsc.load_gather(ublk, [sk, kk16, a_lo, i_id, ju])
                    u1 = plsc.load_gather(ublk, [sk, kk16, a_hi, i_id, ju])
                    v0 = plsc.load_gather(vblk, [sk, kk16, a_lo, i_id, jv])
                    v1 = plsc.load_gather(vblk, [sk, kk16, a_hi, i_id, jv])
                    t = u0 * v0 * w0 + u1 * v1 * w1
                    tbuf[pl.ds((p * QR + k) * L, L)] = t
            s = bias
            for k in range(L):
                s = s + plsc.load_gather(tbuf, [lanes * L + k])
            acc[pl.ds(g * L, L)] = s
            return carry

        lax.fori_loop(0, NG, group, 0)

        for k in range(QR):
            drain(0, k)

        pltpu.sync_copy(acc, out_hbm.at[pl.ds(base, b_per_w)])

    return gmf_sc(user_r, item_r, tut3, tit3, wb)
